# baseline (device time: 38585 ns/iter reference)
import jax
import jax.numpy as jnp
from jax import lax
from jax.experimental import pallas as pl
from jax.experimental.pallas import tpu as pltpu

NCHUNK = 16


def kernel(x, pi):
    _, m, n = x.shape
    half = m // 2
    rows_c = half // NCHUNK

    def body(pi_ref, x_ref, out_ref, ybuf_s, ybuf_r, xbuf_r, obuf,
             ysend_sem, yrecv_sem, xsend_sem, xrecv_sem,
             oy_sem, ox_sem, loc_sem):
        my_x = lax.axis_index("x")
        my_y = lax.axis_index("y")
        tgt_y = pi_ref[my_y]

        @pl.when(tgt_y == my_y)
        def _():
            cp = pltpu.make_async_copy(x_ref, out_ref, loc_sem)
            cp.start()
            cp.wait()

        @pl.when(tgt_y != my_y)
        def _():
            barrier_sem = pltpu.get_barrier_semaphore()
            pl.semaphore_signal(
                barrier_sem, inc=1,
                device_id=(my_x, tgt_y), device_id_type=pl.DeviceIdType.MESH,
            )
            pl.semaphore_signal(
                barrier_sem, inc=1,
                device_id=(1 - my_x, my_y), device_id_type=pl.DeviceIdType.MESH,
            )
            pl.semaphore_wait(barrier_sem, 2)

            h0 = my_x * half
            g0 = (1 - my_x) * half

            y_rdma = []
            for c in range(NCHUNK):
                ybuf_s[c] = x_ref[0, pl.ds(h0 + c * rows_c, rows_c), :].astype(
                    jnp.bfloat16
                )
                rdma = pltpu.make_async_remote_copy(
                    src_ref=ybuf_s.at[c],
                    dst_ref=ybuf_r.at[c],
                    send_sem=ysend_sem.at[c],
                    recv_sem=yrecv_sem.at[c],
                    device_id=(my_x, tgt_y),
                    device_id_type=pl.DeviceIdType.MESH,
                )
                rdma.start()
                y_rdma.append(rdma)

            def drain_chunk(row0, src_buf, c, sem_arr, copies):
                rows = pl.ds(row0 + c * rows_c, rows_c)
                obuf[rows, :] = src_buf[c].astype(jnp.float32)
                cp = pltpu.make_async_copy(
                    obuf.at[rows, :], out_ref.at[0, rows, :], sem_arr.at[c]
                )
                cp.start()
                copies.append(cp)

            x_rdma = []
            ocopies = []
            for c in range(NCHUNK):
                y_rdma[c].wait_recv()
                rdma = pltpu.make_async_remote_copy(
                    src_ref=ybuf_r.at[c],
                    dst_ref=xbuf_r.at[c],
                    send_sem=xsend_sem.at[c],
                    recv_sem=xrecv_sem.at[c],
                    device_id=(1 - my_x, my_y),
                    device_id_type=pl.DeviceIdType.MESH,
                )
                rdma.start()
                x_rdma.append(rdma)
                drain_chunk(h0, ybuf_r, c, oy_sem, ocopies)
                if c >= 2:
                    x_rdma[c - 2].wait_recv()
                    drain_chunk(g0, xbuf_r, c - 2, ox_sem, ocopies)

            for c in range(NCHUNK - 2, NCHUNK):
                x_rdma[c].wait_recv()
                drain_chunk(g0, xbuf_r, c, ox_sem, ocopies)

            for cp in ocopies:
                cp.wait()
            for c in range(NCHUNK):
                y_rdma[c].wait_send()
                x_rdma[c].wait_send()

    return pl.pallas_call(
        body,
        out_shape=jax.ShapeDtypeStruct(x.shape, x.dtype),
        in_specs=[
            pl.BlockSpec(memory_space=pltpu.SMEM),
            pl.BlockSpec(memory_space=pltpu.VMEM),
        ],
        out_specs=pl.BlockSpec(memory_space=pl.ANY),
        scratch_shapes=[
            pltpu.VMEM((NCHUNK, rows_c, n), jnp.bfloat16),
            pltpu.VMEM((NCHUNK, rows_c, n), jnp.bfloat16),
            pltpu.VMEM((NCHUNK, rows_c, n), jnp.bfloat16),
            pltpu.VMEM((m, n), jnp.float32),
            pltpu.SemaphoreType.DMA((NCHUNK,)),
            pltpu.SemaphoreType.DMA((NCHUNK,)),
            pltpu.SemaphoreType.DMA((NCHUNK,)),
            pltpu.SemaphoreType.DMA((NCHUNK,)),
            pltpu.SemaphoreType.DMA((NCHUNK,)),
            pltpu.SemaphoreType.DMA((NCHUNK,)),
            pltpu.SemaphoreType.DMA,
        ],
        compiler_params=pltpu.CompilerParams(collective_id=0),
    )(pi, x)
